# baseline (speedup 1.0000x reference)
import jax
import jax.numpy as jnp
from jax.experimental import pallas as pl
from jax.experimental.pallas import tpu as pltpu

_SEQ_LENS = (2048, 1792, 1536, 1280, 1024, 768, 512, 256)
_D = 1024
_MAXL = 2048
_MAXPAD = _MAXL - min(_SEQ_LENS)


def _pad_body(*refs):
    xs = refs[:8]
    out = refs[8]
    zbuf = refs[9]
    sems = refs[10]

    zbuf[...] = jnp.zeros_like(zbuf)

    copies = []
    n = 0
    for k, L in enumerate(_SEQ_LENS):
        c = pltpu.make_async_copy(xs[k], out.at[k, pl.ds(0, L), :], sems.at[n])
        c.start()
        copies.append(c)
        n += 1
    for k, L in enumerate(_SEQ_LENS):
        pad = _MAXL - L
        if pad == 0:
            continue
        c = pltpu.make_async_copy(
            zbuf.at[pl.ds(0, pad), :], out.at[k, pl.ds(L, pad), :], sems.at[n]
        )
        c.start()
        copies.append(c)
        n += 1
    for c in copies:
        c.wait()


def kernel(x0, x1, x2, x3, x4, x5, x6, x7):
    return pl.pallas_call(
        _pad_body,
        out_shape=jax.ShapeDtypeStruct((8, _MAXL, _D), jnp.float32),
        in_specs=[pl.BlockSpec(memory_space=pl.ANY)] * 8,
        out_specs=pl.BlockSpec(memory_space=pl.ANY),
        scratch_shapes=[
            pltpu.VMEM((_MAXPAD, _D), jnp.float32),
            pltpu.SemaphoreType.DMA((15,)),
        ],
    )(x0, x1, x2, x3, x4, x5, x6, x7)


# speedup vs baseline: 17.5000x; 17.5000x over previous
import functools

import jax
import jax.numpy as jnp
from jax import lax
from jax.experimental import pallas as pl
from jax.experimental.pallas import tpu as pltpu
from jax.experimental.pallas import tpu_sc as plsc

_SEQ_LENS = (2048, 1792, 1536, 1280, 1024, 768, 512, 256)
_D = 1024
_MAXL = 2048
_NC = 2
_NS = 16
_CH = 32
_CHUNKS = 16


def _worker(seq, q, x, out, zbuf, dbuf0, dbuf1, rsems, zwsem):
    L = _SEQ_LENS[seq]
    c = jnp.clip(L - 512 * q, 0, 512)
    npairs = c // (2 * _CH)
    nchunks = 2 * npairs
    base = seq * _MAXL + q * 512

    def zfire(k, _):
        pltpu.async_copy(zbuf, out.at[pl.ds(base + k * _CH, _CH), :], zwsem)
        return _

    lax.fori_loop(nchunks, _CHUNKS, zfire, None)

    @pl.when(npairs > 0)
    def _():
        pltpu.async_copy(x.at[pl.ds(q * 512, _CH), :], dbuf0, rsems.at[0])

    def pair(p, _):
        r0 = q * 512 + p * 2 * _CH
        pltpu.async_copy(x.at[pl.ds(r0 + _CH, _CH), :], dbuf1, rsems.at[1])
        pltpu.make_async_copy(x.at[pl.ds(r0, _CH), :], dbuf0, rsems.at[0]).wait()
        pltpu.sync_copy(dbuf0, out.at[pl.ds(base + p * 2 * _CH, _CH), :])

        @pl.when(p + 1 < npairs)
        def _():
            pltpu.async_copy(x.at[pl.ds(r0 + 2 * _CH, _CH), :], dbuf0, rsems.at[0])

        pltpu.make_async_copy(x.at[pl.ds(r0 + _CH, _CH), :], dbuf1, rsems.at[1]).wait()
        pltpu.sync_copy(dbuf1, out.at[pl.ds(base + p * 2 * _CH + _CH, _CH), :])
        return _

    lax.fori_loop(0, npairs, pair, None)

    def zdrain(k, _):
        pltpu.make_async_copy(zbuf, out.at[pl.ds(base, _CH), :], zwsem).wait()
        return _

    lax.fori_loop(nchunks, _CHUNKS, zdrain, None)


def _pad_body(x0, x1, x2, x3, x4, x5, x6, x7, zsrc, out, zbuf, dbuf0, dbuf1,
              rsems, zwsem):
    xs = (x0, x1, x2, x3, x4, x5, x6, x7)
    w = lax.axis_index("s") * _NC + lax.axis_index("c")
    i = w // 4
    q = w % 4
    pltpu.sync_copy(zsrc, zbuf)
    for seq in range(8):

        @pl.when(i == seq)
        def _(seq=seq):
            _worker(seq, q, xs[seq], out, zbuf, dbuf0, dbuf1, rsems, zwsem)


@functools.partial(
    pl.kernel,
    out_type=jax.ShapeDtypeStruct((8 * _MAXL, _D), jnp.float32),
    mesh=plsc.VectorSubcoreMesh(core_axis_name="c", subcore_axis_name="s"),
    scratch_types=[
        pltpu.VMEM((_CH, _D), jnp.float32),
        pltpu.VMEM((_CH, _D), jnp.float32),
        pltpu.VMEM((_CH, _D), jnp.float32),
        pltpu.SemaphoreType.DMA((2,)),
        pltpu.SemaphoreType.DMA,
    ],
)
def _pad_sc(*refs):
    _pad_body(*refs)


def kernel(x0, x1, x2, x3, x4, x5, x6, x7):
    zsrc = jnp.zeros((_CH, _D), jnp.float32)
    out = _pad_sc(x0, x1, x2, x3, x4, x5, x6, x7, zsrc)
    return out.reshape(8, _MAXL, _D)
